# baseline (speedup 1.0000x reference)
import jax
import jax.numpy as jnp
from jax import lax
from jax.experimental import pallas as pl

_MULT_SIZE = 1.2


def _pad_body(species_ref, natoms_ref, batch_ref, coords_ref, cells_ref,
              species_out_ref, natoms_out_ref, batch_out_ref, coords_out_ref,
              cells_out_ref, true_atoms_ref, true_sys_ref):
    nat = species_ref.shape[0]
    nsys = natoms_ref.shape[0]
    pad_nat = species_out_ref.shape[0]
    add = pad_nat - nat

    s = species_ref[...]
    species_out_ref[0:nat] = s
    species_out_ref[nat:pad_nat] = jnp.full((add,), -1, species_ref.dtype)
    true_atoms_ref[0:nat] = s > 0
    true_atoms_ref[nat:pad_nat] = jnp.zeros((add,), jnp.bool_)

    batch_out_ref[0:nat] = batch_ref[...]
    batch_out_ref[nat:pad_nat] = jnp.full((add,), nsys, batch_ref.dtype)

    nflat = coords_ref.shape[0]
    pad_flat = coords_out_ref.shape[0]
    coords_out_ref[0:nflat] = coords_ref[...]
    coords_out_ref[nflat:pad_flat] = jnp.zeros((pad_flat - nflat,),
                                               coords_ref.dtype)

    natoms_out_ref[0:nsys] = natoms_ref[...]
    natoms_out_ref[nsys:nsys + 1] = jnp.full((1,), add, natoms_ref.dtype)

    cells_out_ref[0:nsys] = cells_ref[...]
    i = lax.broadcasted_iota(jnp.int32, (1, 3, 3), 1)
    j = lax.broadcasted_iota(jnp.int32, (1, 3, 3), 2)
    cells_out_ref[nsys:nsys + 1] = (i == j).astype(cells_ref.dtype)

    true_sys_ref[0:nsys] = jnp.ones((nsys,), jnp.bool_)
    true_sys_ref[nsys:nsys + 1] = jnp.zeros((1,), jnp.bool_)


def kernel(species, natoms, batch_index, coordinates, cells):
    nat = species.shape[0]
    nsys = natoms.shape[0]
    pad_nat = int(_MULT_SIZE * nat) + 1
    ndim = coordinates.shape[1]

    out_shape = (
        jax.ShapeDtypeStruct((pad_nat,), species.dtype),
        jax.ShapeDtypeStruct((nsys + 1,), natoms.dtype),
        jax.ShapeDtypeStruct((pad_nat,), batch_index.dtype),
        jax.ShapeDtypeStruct((pad_nat * ndim,), coordinates.dtype),
        jax.ShapeDtypeStruct((nsys + 1,) + cells.shape[1:], cells.dtype),
        jax.ShapeDtypeStruct((pad_nat,), jnp.bool_),
        jax.ShapeDtypeStruct((nsys + 1,), jnp.bool_),
    )
    (species_out, natoms_out, batch_out, coords_flat, cells_out,
     true_atoms, true_sys) = pl.pallas_call(_pad_body, out_shape=out_shape)(
        species, natoms, batch_index, coordinates.reshape(-1), cells)
    return (species_out, natoms_out, batch_out,
            coords_flat.reshape(pad_nat, ndim), cells_out, true_atoms, true_sys)


# speedup vs baseline: 10.1728x; 10.1728x over previous
import jax
import jax.numpy as jnp
from jax import lax
from jax.experimental import pallas as pl

_MULT_SIZE = 1.2


def _pad_body(species_ref, natoms_ref, batch_ref, coordsT_ref,
              species_out_ref, natoms_out_ref, batch_out_ref, coordsT_out_ref,
              true_atoms_ref):
    nat = species_ref.shape[0]
    nsys = natoms_ref.shape[0]
    pad_nat = species_out_ref.shape[0]
    add = pad_nat - nat

    s = species_ref[...]
    species_out_ref[0:nat] = s
    species_out_ref[nat:pad_nat] = jnp.full((add,), -1, species_ref.dtype)
    true_atoms_ref[0:nat] = s > 0
    true_atoms_ref[nat:pad_nat] = jnp.zeros((add,), jnp.bool_)

    batch_out_ref[0:nat] = batch_ref[...]
    batch_out_ref[nat:pad_nat] = jnp.full((add,), nsys, batch_ref.dtype)

    coordsT_out_ref[:, 0:nat] = coordsT_ref[...]
    coordsT_out_ref[:, nat:pad_nat] = jnp.zeros(
        (coordsT_ref.shape[0], add), coordsT_ref.dtype)

    natoms_out_ref[0:nsys] = natoms_ref[...]
    natoms_out_ref[nsys:nsys + 1] = jnp.full((1,), add, natoms_ref.dtype)


def kernel(species, natoms, batch_index, coordinates, cells):
    nat = species.shape[0]
    nsys = natoms.shape[0]
    pad_nat = int(_MULT_SIZE * nat) + 1
    ndim = coordinates.shape[1]

    out_shape = (
        jax.ShapeDtypeStruct((pad_nat,), species.dtype),
        jax.ShapeDtypeStruct((nsys + 1,), natoms.dtype),
        jax.ShapeDtypeStruct((pad_nat,), batch_index.dtype),
        jax.ShapeDtypeStruct((ndim, pad_nat), coordinates.dtype),
        jax.ShapeDtypeStruct((pad_nat,), jnp.bool_),
    )
    (species_out, natoms_out, batch_out, coordsT_out,
     true_atoms) = pl.pallas_call(_pad_body, out_shape=out_shape)(
        species, natoms, batch_index, coordinates.T)

    cells_out = jnp.concatenate(
        [cells, jnp.eye(cells.shape[1], dtype=cells.dtype)[None, :, :]], axis=0)
    true_sys = jnp.arange(nsys + 1) < nsys
    return (species_out, natoms_out, batch_out, coordsT_out.T, cells_out,
            true_atoms, true_sys)
